# Initial kernel scaffold; baseline (speedup 1.0000x reference)
import jax
import jax.numpy as jnp
from jax.experimental import pallas as pl


def kernel(d_features, g_features, M1_mirna_dis, M2_gene_dis, edge_index, W1, W2, W3, W4, Wd, bd, Wg, bg, Ws1, Wn1, b1, Ws2, Wn2, b2):
    raise NotImplementedError("write your pallas kernel here")


# speedup vs baseline: 3.4296x; 3.4296x over previous
import functools

import jax
import jax.numpy as jnp
from jax import lax
from jax.experimental import pallas as pl
from jax.experimental.pallas import tpu as pltpu
from jax.experimental.pallas import tpu_sc as plsc

ND = 4000
NG = 6000
NN = ND + NG
NE = 320000
EMB = 128
HID = 64

NC = 2
NS = 16
NW = NC * NS

NJ = 4
SLICE = 128
SUP = NJ * SLICE
NSUP = 20
EPW = SUP * NSUP
NE_PAD = EPW * NW
OPS_PER_W = EPW // SLICE

R_PAD = 10240
RPT = R_PAD // NS


def _dis_body(d_ref, m1_ref, wd_ref, bd_ref, w1_ref, ws1_ref, wn1_ref,
              s1_ref, t1_ref):
    rep = 0.9 * (jnp.dot(d_ref[...], wd_ref[...],
                         preferred_element_type=jnp.float32) + bd_ref[...])
    rep = rep + 0.1 * lax.dot_general(
        m1_ref[...], w1_ref[...], (((0,), (0,)), ((), ())),
        preferred_element_type=jnp.float32)
    s1_ref[...] = jnp.dot(rep, ws1_ref[...], preferred_element_type=jnp.float32)
    t1_ref[...] = jnp.dot(rep, wn1_ref[...], preferred_element_type=jnp.float32)


def _gen_body(g_ref, m2_ref, wg_ref, bg_ref, w4_ref, ws1_ref, wn1_ref,
              s1_ref, t1_ref):
    rep = 0.9 * (jnp.dot(g_ref[...], wg_ref[...],
                         preferred_element_type=jnp.float32) + bg_ref[...])
    rep = rep + 0.1 * jnp.dot(m2_ref[...], w4_ref[...],
                              preferred_element_type=jnp.float32)
    s1_ref[...] = jnp.dot(rep, ws1_ref[...], preferred_element_type=jnp.float32)
    t1_ref[...] = jnp.dot(rep, wn1_ref[...], preferred_element_type=jnp.float32)


def _combine1_body(s1_ref, acc_ref, deg_ref, b1_ref, wn2_ref, ws2_ref,
                   t2_ref, s2_ref):
    agg = acc_ref[0] + acc_ref[1]
    deg = deg_ref[0, :, 0:1] + deg_ref[1, :, 0:1]
    hn = agg / jnp.maximum(deg, 1.0)
    h1 = jnp.maximum(s1_ref[...] + hn + b1_ref[...], 0.0)
    t2_ref[...] = jnp.dot(h1, wn2_ref[...], preferred_element_type=jnp.float32)
    s2_ref[...] = jnp.dot(h1, ws2_ref[...], preferred_element_type=jnp.float32)


def _combine2_body(s2_ref, acc_ref, deg_ref, b2_ref, out_ref):
    agg = acc_ref[0] + acc_ref[1]
    deg = deg_ref[0, :, 0:1] + deg_ref[1, :, 0:1]
    out_ref[...] = s2_ref[...] + agg / jnp.maximum(deg, 1.0) + b2_ref[...]


def _dis_call(d, m1, wd, bd, w1, ws1, wn1):
    blk = 512
    grid = (ND + blk - 1) // blk
    return pl.pallas_call(
        _dis_body,
        grid=(grid,),
        in_specs=[
            pl.BlockSpec((blk, 383), lambda i: (i, 0)),
            pl.BlockSpec((2000, blk), lambda i: (0, i)),
            pl.BlockSpec((383, EMB), lambda i: (0, 0)),
            pl.BlockSpec((1, EMB), lambda i: (0, 0)),
            pl.BlockSpec((2000, EMB), lambda i: (0, 0)),
            pl.BlockSpec((EMB, HID), lambda i: (0, 0)),
            pl.BlockSpec((EMB, HID), lambda i: (0, 0)),
        ],
        out_specs=[
            pl.BlockSpec((blk, HID), lambda i: (i, 0)),
            pl.BlockSpec((blk, HID), lambda i: (i, 0)),
        ],
        out_shape=[
            jax.ShapeDtypeStruct((ND, HID), jnp.float32),
            jax.ShapeDtypeStruct((ND, HID), jnp.float32),
        ],
    )(d, m1, wd, bd, w1, ws1, wn1)


def _gen_call(g, m2, wg, bg, w4, ws1, wn1):
    blk = 256
    grid = (NG + blk - 1) // blk
    return pl.pallas_call(
        _gen_body,
        grid=(grid,),
        in_specs=[
            pl.BlockSpec((blk, 4395), lambda i: (i, 0)),
            pl.BlockSpec((blk, 4000), lambda i: (i, 0)),
            pl.BlockSpec((4395, EMB), lambda i: (0, 0)),
            pl.BlockSpec((1, EMB), lambda i: (0, 0)),
            pl.BlockSpec((4000, EMB), lambda i: (0, 0)),
            pl.BlockSpec((EMB, HID), lambda i: (0, 0)),
            pl.BlockSpec((EMB, HID), lambda i: (0, 0)),
        ],
        out_specs=[
            pl.BlockSpec((blk, HID), lambda i: (i, 0)),
            pl.BlockSpec((blk, HID), lambda i: (i, 0)),
        ],
        out_shape=[
            jax.ShapeDtypeStruct((NG, HID), jnp.float32),
            jax.ShapeDtypeStruct((NG, HID), jnp.float32),
        ],
    )(g, m2, wg, bg, w4, ws1, wn1)


def _combine1_call(s1, acc, deg, b1, wn2, ws2):
    blk = 512
    grid = (NN + blk - 1) // blk
    return pl.pallas_call(
        _combine1_body,
        grid=(grid,),
        in_specs=[
            pl.BlockSpec((blk, HID), lambda i: (i, 0)),
            pl.BlockSpec((NC, blk, HID), lambda i: (0, i, 0)),
            pl.BlockSpec((NC, blk, 16), lambda i: (0, i, 0)),
            pl.BlockSpec((1, HID), lambda i: (0, 0)),
            pl.BlockSpec((HID, HID), lambda i: (0, 0)),
            pl.BlockSpec((HID, HID), lambda i: (0, 0)),
        ],
        out_specs=[
            pl.BlockSpec((blk, HID), lambda i: (i, 0)),
            pl.BlockSpec((blk, HID), lambda i: (i, 0)),
        ],
        out_shape=[
            jax.ShapeDtypeStruct((NN, HID), jnp.float32),
            jax.ShapeDtypeStruct((NN, HID), jnp.float32),
        ],
    )(s1, acc, deg, b1, wn2, ws2)


def _combine2_call(s2, acc, deg, b2):
    blk = 512
    grid = (NN + blk - 1) // blk
    return pl.pallas_call(
        _combine2_body,
        grid=(grid,),
        in_specs=[
            pl.BlockSpec((blk, HID), lambda i: (i, 0)),
            pl.BlockSpec((NC, blk, HID), lambda i: (0, i, 0)),
            pl.BlockSpec((NC, blk, 16), lambda i: (0, i, 0)),
            pl.BlockSpec((1, HID), lambda i: (0, 0)),
        ],
        out_specs=pl.BlockSpec((blk, HID), lambda i: (i, 0)),
        out_shape=jax.ShapeDtypeStruct((NN, HID), jnp.float32),
    )(s2, acc, deg, b2)


@functools.lru_cache(maxsize=None)
def _sc_agg_deg_kernel():
    return functools.partial(
        pl.kernel,
        mesh=plsc.VectorSubcoreMesh(core_axis_name="c", subcore_axis_name="s"),
        compiler_params=pltpu.CompilerParams(use_tc_tiling_on_sc=False),
        out_type=[
            jax.ShapeDtypeStruct((NC, R_PAD, HID), jnp.float32),
            jax.ShapeDtypeStruct((NC, R_PAD, 16), jnp.float32),
        ],
        scratch_types=[
            pltpu.VMEM((NJ, SLICE), jnp.int32),
            pltpu.VMEM((NJ, SLICE), jnp.int32),
            pltpu.VMEM((NJ, SLICE, HID), jnp.float32),
            pltpu.VMEM((SLICE, 16), jnp.float32),
            pltpu.VMEM_SHARED((R_PAD, HID), jnp.float32),
            pltpu.VMEM_SHARED((R_PAD, 16), jnp.float32),
            pltpu.SemaphoreType.DMA,
        ],
    )(_sc_agg_deg_body)


def _sc_agg_deg_body(t_hbm, src_hbm, dst_hbm, za_hbm, zd_hbm, on_hbm,
                     acc_out, deg_out,
                     idx_s, idx_d, rows, ones_v, acc_sh, deg_sh, sem):
    c = lax.axis_index("c")
    s = lax.axis_index("s")
    wid = s * NC + c
    pltpu.sync_copy(za_hbm, acc_sh.at[pl.ds(s * RPT, RPT)])
    pltpu.sync_copy(zd_hbm, deg_sh.at[pl.ds(s * RPT, RPT)])
    pltpu.sync_copy(on_hbm, ones_v)
    plsc.subcore_barrier()

    opbase = wid * OPS_PER_W

    def step(u, carry):
        row0 = opbase + u * NJ
        pltpu.sync_copy(src_hbm.at[pl.ds(row0, NJ)], idx_s)
        pltpu.sync_copy(dst_hbm.at[pl.ds(row0, NJ)], idx_d)
        cps = [pltpu.async_copy(t_hbm.at[idx_s.at[j]], rows.at[j], sem)
               for j in range(NJ)]
        for cp in cps:
            cp.wait()
        for j in range(NJ):
            pltpu.sync_copy(rows.at[j], acc_sh.at[idx_d.at[j]], add=True)
            pltpu.sync_copy(ones_v, deg_sh.at[idx_d.at[j]], add=True)
        return carry

    lax.fori_loop(0, NSUP, step, 0)
    plsc.subcore_barrier()
    pltpu.sync_copy(acc_sh.at[pl.ds(s * RPT, RPT)],
                    acc_out.at[c, pl.ds(s * RPT, RPT)])
    pltpu.sync_copy(deg_sh.at[pl.ds(s * RPT, RPT)],
                    deg_out.at[c, pl.ds(s * RPT, RPT)])


@functools.lru_cache(maxsize=None)
def _sc_agg_kernel():
    return functools.partial(
        pl.kernel,
        mesh=plsc.VectorSubcoreMesh(core_axis_name="c", subcore_axis_name="s"),
        compiler_params=pltpu.CompilerParams(use_tc_tiling_on_sc=False),
        out_type=jax.ShapeDtypeStruct((NC, R_PAD, HID), jnp.float32),
        scratch_types=[
            pltpu.VMEM((NJ, SLICE), jnp.int32),
            pltpu.VMEM((NJ, SLICE), jnp.int32),
            pltpu.VMEM((NJ, SLICE, HID), jnp.float32),
            pltpu.VMEM_SHARED((R_PAD, HID), jnp.float32),
            pltpu.SemaphoreType.DMA,
        ],
    )(_sc_agg_body)


def _sc_agg_body(t_hbm, src_hbm, dst_hbm, za_hbm,
                 acc_out,
                 idx_s, idx_d, rows, acc_sh, sem):
    c = lax.axis_index("c")
    s = lax.axis_index("s")
    wid = s * NC + c
    pltpu.sync_copy(za_hbm, acc_sh.at[pl.ds(s * RPT, RPT)])
    plsc.subcore_barrier()

    opbase = wid * OPS_PER_W

    def step(u, carry):
        row0 = opbase + u * NJ
        pltpu.sync_copy(src_hbm.at[pl.ds(row0, NJ)], idx_s)
        pltpu.sync_copy(dst_hbm.at[pl.ds(row0, NJ)], idx_d)
        cps = [pltpu.async_copy(t_hbm.at[idx_s.at[j]], rows.at[j], sem)
               for j in range(NJ)]
        for cp in cps:
            cp.wait()
        for j in range(NJ):
            pltpu.sync_copy(rows.at[j], acc_sh.at[idx_d.at[j]], add=True)
        return carry

    lax.fori_loop(0, NSUP, step, 0)
    plsc.subcore_barrier()
    pltpu.sync_copy(acc_sh.at[pl.ds(s * RPT, RPT)],
                    acc_out.at[c, pl.ds(s * RPT, RPT)])


def kernel(d_features, g_features, M1_mirna_dis, M2_gene_dis, edge_index,
           W1, W2, W3, W4, Wd, bd, Wg, bg, Ws1, Wn1, b1, Ws2, Wn2, b2):
    src = edge_index[0]
    dst = edge_index[1]
    npad = NE_PAD - NE
    src_p = jnp.concatenate([src, jnp.zeros((npad,), jnp.int32)])
    dst_p = jnp.concatenate([dst, jnp.full((npad,), NN, jnp.int32)])
    src2 = src_p.reshape(NE_PAD // SLICE, SLICE)
    dst2 = dst_p.reshape(NE_PAD // SLICE, SLICE)
    za = jnp.zeros((RPT, HID), jnp.float32)
    zd = jnp.zeros((RPT, 16), jnp.float32)
    on = jnp.ones((SLICE, 16), jnp.float32)

    bd2 = bd.reshape(1, EMB)
    bg2 = bg.reshape(1, EMB)
    b12 = b1.reshape(1, HID)
    b22 = b2.reshape(1, HID)

    s1d, t1d = _dis_call(d_features, M1_mirna_dis, Wd, bd2, W1, Ws1, Wn1)
    s1g, t1g = _gen_call(g_features, M2_gene_dis, Wg, bg2, W4, Ws1, Wn1)
    s1 = jnp.concatenate([s1d, s1g], axis=0)
    t1 = jnp.concatenate([t1d, t1g], axis=0)

    acc1, deg = _sc_agg_deg_kernel()(t1, src2, dst2, za, zd, on)
    t2, s2 = _combine1_call(s1, acc1, deg, b12, Wn2, Ws2)
    acc2 = _sc_agg_kernel()(t2, src2, dst2, za)
    return _combine2_call(s2, acc2, deg, b22)


# speedup vs baseline: 3.5839x; 1.0450x over previous
import functools

import jax
import jax.numpy as jnp
from jax import lax
from jax.experimental import pallas as pl
from jax.experimental.pallas import tpu as pltpu
from jax.experimental.pallas import tpu_sc as plsc

ND = 4000
NG = 6000
NN = ND + NG
NE = 320000
EMB = 128
HID = 64
TW1 = HID + 16

NC = 2
NS = 16
NW = NC * NS

SLICE = 128
OPS_PER_W = 80
EPW = OPS_PER_W * SLICE
NE_PAD = EPW * NW

R_PAD = 10240
RPT = R_PAD // NS


def _dis_body(d_ref, m1_ref, wd_ref, bd_ref, w1_ref, ws1_ref, wn1_ref,
              s1_ref, t1_ref):
    rep = 0.9 * (jnp.dot(d_ref[...], wd_ref[...],
                         preferred_element_type=jnp.float32) + bd_ref[...])
    rep = rep + 0.1 * lax.dot_general(
        m1_ref[...], w1_ref[...], (((0,), (0,)), ((), ())),
        preferred_element_type=jnp.float32)
    s1_ref[...] = jnp.dot(rep, ws1_ref[...], preferred_element_type=jnp.float32)
    t1 = jnp.dot(rep, wn1_ref[...], preferred_element_type=jnp.float32)
    t1_ref[...] = jnp.concatenate(
        [t1, jnp.ones((t1.shape[0], 16), jnp.float32)], axis=1)


def _gen_body(g_ref, m2_ref, wg_ref, bg_ref, w4_ref, ws1_ref, wn1_ref,
              s1_ref, t1_ref):
    rep = 0.9 * (jnp.dot(g_ref[...], wg_ref[...],
                         preferred_element_type=jnp.float32) + bg_ref[...])
    rep = rep + 0.1 * jnp.dot(m2_ref[...], w4_ref[...],
                              preferred_element_type=jnp.float32)
    s1_ref[...] = jnp.dot(rep, ws1_ref[...], preferred_element_type=jnp.float32)
    t1 = jnp.dot(rep, wn1_ref[...], preferred_element_type=jnp.float32)
    t1_ref[...] = jnp.concatenate(
        [t1, jnp.ones((t1.shape[0], 16), jnp.float32)], axis=1)


def _combine1_body(s1_ref, acc_ref, b1_ref, wn2_ref, ws2_ref,
                   t2_ref, s2_ref):
    a = acc_ref[0] + acc_ref[1]
    agg = a[:, :HID]
    deg = a[:, HID:HID + 1]
    hn = agg / jnp.maximum(deg, 1.0)
    h1 = jnp.maximum(s1_ref[...] + hn + b1_ref[...], 0.0)
    t2_ref[...] = jnp.dot(h1, wn2_ref[...], preferred_element_type=jnp.float32)
    s2_ref[...] = jnp.dot(h1, ws2_ref[...], preferred_element_type=jnp.float32)


def _combine2_body(s2_ref, acc_ref, acc1_ref, b2_ref, out_ref):
    agg = acc_ref[0] + acc_ref[1]
    deg = acc1_ref[0, :, HID:HID + 1] + acc1_ref[1, :, HID:HID + 1]
    out_ref[...] = s2_ref[...] + agg / jnp.maximum(deg, 1.0) + b2_ref[...]


def _dis_call(d, m1, wd, bd, w1, ws1, wn1):
    blk = 512
    grid = (ND + blk - 1) // blk
    return pl.pallas_call(
        _dis_body,
        grid=(grid,),
        in_specs=[
            pl.BlockSpec((blk, 383), lambda i: (i, 0)),
            pl.BlockSpec((2000, blk), lambda i: (0, i)),
            pl.BlockSpec((383, EMB), lambda i: (0, 0)),
            pl.BlockSpec((1, EMB), lambda i: (0, 0)),
            pl.BlockSpec((2000, EMB), lambda i: (0, 0)),
            pl.BlockSpec((EMB, HID), lambda i: (0, 0)),
            pl.BlockSpec((EMB, HID), lambda i: (0, 0)),
        ],
        out_specs=[
            pl.BlockSpec((blk, HID), lambda i: (i, 0)),
            pl.BlockSpec((blk, TW1), lambda i: (i, 0)),
        ],
        out_shape=[
            jax.ShapeDtypeStruct((ND, HID), jnp.float32),
            jax.ShapeDtypeStruct((ND, TW1), jnp.float32),
        ],
    )(d, m1, wd, bd, w1, ws1, wn1)


def _gen_call(g, m2, wg, bg, w4, ws1, wn1):
    blk = 256
    grid = (NG + blk - 1) // blk
    return pl.pallas_call(
        _gen_body,
        grid=(grid,),
        in_specs=[
            pl.BlockSpec((blk, 4395), lambda i: (i, 0)),
            pl.BlockSpec((blk, 4000), lambda i: (i, 0)),
            pl.BlockSpec((4395, EMB), lambda i: (0, 0)),
            pl.BlockSpec((1, EMB), lambda i: (0, 0)),
            pl.BlockSpec((4000, EMB), lambda i: (0, 0)),
            pl.BlockSpec((EMB, HID), lambda i: (0, 0)),
            pl.BlockSpec((EMB, HID), lambda i: (0, 0)),
        ],
        out_specs=[
            pl.BlockSpec((blk, HID), lambda i: (i, 0)),
            pl.BlockSpec((blk, TW1), lambda i: (i, 0)),
        ],
        out_shape=[
            jax.ShapeDtypeStruct((NG, HID), jnp.float32),
            jax.ShapeDtypeStruct((NG, TW1), jnp.float32),
        ],
    )(g, m2, wg, bg, w4, ws1, wn1)


def _combine1_call(s1, acc, b1, wn2, ws2):
    blk = 512
    grid = (NN + blk - 1) // blk
    return pl.pallas_call(
        _combine1_body,
        grid=(grid,),
        in_specs=[
            pl.BlockSpec((blk, HID), lambda i: (i, 0)),
            pl.BlockSpec((NC, blk, TW1), lambda i: (0, i, 0)),
            pl.BlockSpec((1, HID), lambda i: (0, 0)),
            pl.BlockSpec((HID, HID), lambda i: (0, 0)),
            pl.BlockSpec((HID, HID), lambda i: (0, 0)),
        ],
        out_specs=[
            pl.BlockSpec((blk, HID), lambda i: (i, 0)),
            pl.BlockSpec((blk, HID), lambda i: (i, 0)),
        ],
        out_shape=[
            jax.ShapeDtypeStruct((NN, HID), jnp.float32),
            jax.ShapeDtypeStruct((NN, HID), jnp.float32),
        ],
    )(s1, acc, b1, wn2, ws2)


def _combine2_call(s2, acc2, acc1, b2):
    blk = 512
    grid = (NN + blk - 1) // blk
    return pl.pallas_call(
        _combine2_body,
        grid=(grid,),
        in_specs=[
            pl.BlockSpec((blk, HID), lambda i: (i, 0)),
            pl.BlockSpec((NC, blk, HID), lambda i: (0, i, 0)),
            pl.BlockSpec((NC, blk, TW1), lambda i: (0, i, 0)),
            pl.BlockSpec((1, HID), lambda i: (0, 0)),
        ],
        out_specs=pl.BlockSpec((blk, HID), lambda i: (i, 0)),
        out_shape=jax.ShapeDtypeStruct((NN, HID), jnp.float32),
    )(s2, acc2, acc1, b2)


def _sc_body(tw, nj, *refs):
    (t_hbm, src_hbm, dst_hbm, za_hbm,
     acc_out,
     idx_s, idx_d, rows, acc_sh, sem_g, sem_s) = refs
    nsup = OPS_PER_W // nj
    c = lax.axis_index("c")
    s = lax.axis_index("s")
    wid = s * NC + c
    opbase = wid * OPS_PER_W

    pltpu.sync_copy(za_hbm, acc_sh.at[pl.ds(s * RPT, RPT)])
    pltpu.sync_copy(src_hbm.at[pl.ds(opbase, OPS_PER_W)], idx_s)
    pltpu.sync_copy(dst_hbm.at[pl.ds(opbase, OPS_PER_W)], idx_d)
    plsc.subcore_barrier()

    def fire_gathers(u, b):
        for j in range(nj):
            pltpu.async_copy(t_hbm.at[idx_s.at[u * nj + j]],
                             rows.at[b, j], sem_g)

    def drain(sem):
        for _ in range(nj):
            pltpu.make_async_copy(t_hbm.at[pl.ds(0, SLICE)],
                                  rows.at[0, 0], sem).wait()

    fire_gathers(0, 0)

    def step(u, carry):
        b = lax.rem(u, 2)
        nb = 1 - b
        drain(sem_g)

        @pl.when(u >= 1)
        def _():
            drain(sem_s)

        @pl.when(u <= nsup - 2)
        def _():
            fire_gathers(u + 1, nb)

        for j in range(nj):
            pltpu.async_copy(rows.at[b, j], acc_sh.at[idx_d.at[u * nj + j]],
                             sem_s, add=True)
        return carry

    lax.fori_loop(0, nsup, step, 0)
    drain(sem_s)
    plsc.subcore_barrier()
    pltpu.sync_copy(acc_sh.at[pl.ds(s * RPT, RPT)],
                    acc_out.at[c, pl.ds(s * RPT, RPT)])


@functools.lru_cache(maxsize=None)
def _sc_agg_kernel(tw, nj):
    return functools.partial(
        pl.kernel,
        mesh=plsc.VectorSubcoreMesh(core_axis_name="c", subcore_axis_name="s"),
        compiler_params=pltpu.CompilerParams(use_tc_tiling_on_sc=False),
        out_type=jax.ShapeDtypeStruct((NC, R_PAD, tw), jnp.float32),
        scratch_types=[
            pltpu.VMEM((OPS_PER_W, SLICE), jnp.int32),
            pltpu.VMEM((OPS_PER_W, SLICE), jnp.int32),
            pltpu.VMEM((2, nj, SLICE, tw), jnp.float32),
            pltpu.VMEM_SHARED((R_PAD, tw), jnp.float32),
            pltpu.SemaphoreType.DMA,
            pltpu.SemaphoreType.DMA,
        ],
    )(functools.partial(_sc_body, tw, nj))


def kernel(d_features, g_features, M1_mirna_dis, M2_gene_dis, edge_index,
           W1, W2, W3, W4, Wd, bd, Wg, bg, Ws1, Wn1, b1, Ws2, Wn2, b2):
    src = edge_index[0]
    dst = edge_index[1]
    npad = NE_PAD - NE
    src_p = jnp.concatenate([src, jnp.zeros((npad,), jnp.int32)])
    dst_p = jnp.concatenate([dst, jnp.full((npad,), NN, jnp.int32)])
    src2 = src_p.reshape(NE_PAD // SLICE, SLICE)
    dst2 = dst_p.reshape(NE_PAD // SLICE, SLICE)
    za1 = jnp.zeros((RPT, TW1), jnp.float32)
    za2 = jnp.zeros((RPT, HID), jnp.float32)

    bd2 = bd.reshape(1, EMB)
    bg2 = bg.reshape(1, EMB)
    b12 = b1.reshape(1, HID)
    b22 = b2.reshape(1, HID)

    s1d, t1d = _dis_call(d_features, M1_mirna_dis, Wd, bd2, W1, Ws1, Wn1)
    s1g, t1g = _gen_call(g_features, M2_gene_dis, Wg, bg2, W4, Ws1, Wn1)
    s1 = jnp.concatenate([s1d, s1g], axis=0)
    t1 = jnp.concatenate([t1d, t1g], axis=0)

    acc1 = _sc_agg_kernel(TW1, 2)(t1, src2, dst2, za1)
    t2, s2 = _combine1_call(s1, acc1, b12, Wn2, Ws2)
    acc2 = _sc_agg_kernel(HID, 4)(t2, src2, dst2, za2)
    return _combine2_call(s2, acc2, acc1, b22)


# speedup vs baseline: 6.3009x; 1.7581x over previous
import functools

import jax
import jax.numpy as jnp
from jax import lax
from jax.experimental import pallas as pl
from jax.experimental.pallas import tpu as pltpu
from jax.experimental.pallas import tpu_sc as plsc

ND = 4000
NG = 6000
NN = ND + NG
NE = 320000
EMB = 128
HID = 64
TW1 = HID + 16

NC = 2
NS = 16
NW = NC * NS

SLICE = 128
OPS_PER_W = 80
EPW = OPS_PER_W * SLICE
NE_PAD = EPW * NW

R_PAD = 10240
RPT = R_PAD // NS


def _dis_body(d_ref, m1_ref, wd_ref, bd_ref, w1_ref, ws1_ref, wn1_ref,
              s1_ref, t1_ref):
    rep = 0.9 * (jnp.dot(d_ref[...], wd_ref[...],
                         preferred_element_type=jnp.float32) + bd_ref[...])
    rep = rep + 0.1 * lax.dot_general(
        m1_ref[...], w1_ref[...], (((0,), (0,)), ((), ())),
        preferred_element_type=jnp.float32)
    s1_ref[...] = jnp.dot(rep, ws1_ref[...], preferred_element_type=jnp.float32)
    t1 = jnp.dot(rep, wn1_ref[...], preferred_element_type=jnp.float32)
    t1_ref[...] = jnp.concatenate(
        [t1, jnp.ones((t1.shape[0], 16), jnp.float32)], axis=1)


def _gen_body(g_ref, m2_ref, wg_ref, bg_ref, w4_ref, ws1_ref, wn1_ref,
              s1_ref, t1_ref):
    rep = 0.9 * (jnp.dot(g_ref[...], wg_ref[...],
                         preferred_element_type=jnp.float32) + bg_ref[...])
    rep = rep + 0.1 * jnp.dot(m2_ref[...], w4_ref[...],
                              preferred_element_type=jnp.float32)
    s1_ref[...] = jnp.dot(rep, ws1_ref[...], preferred_element_type=jnp.float32)
    t1 = jnp.dot(rep, wn1_ref[...], preferred_element_type=jnp.float32)
    t1_ref[...] = jnp.concatenate(
        [t1, jnp.ones((t1.shape[0], 16), jnp.float32)], axis=1)


def _combine1_body(s1_ref, acc_ref, b1_ref, wn2_ref, ws2_ref,
                   t2_ref, s2_ref):
    a = acc_ref[0] + acc_ref[1]
    agg = a[:, :HID]
    deg = a[:, HID:HID + 1]
    hn = agg / jnp.maximum(deg, 1.0)
    h1 = jnp.maximum(s1_ref[...] + hn + b1_ref[...], 0.0)
    t2_ref[...] = jnp.dot(h1, wn2_ref[...], preferred_element_type=jnp.float32)
    s2_ref[...] = jnp.dot(h1, ws2_ref[...], preferred_element_type=jnp.float32)


def _combine2_body(s2_ref, acc_ref, acc1_ref, b2_ref, out_ref):
    agg = acc_ref[0] + acc_ref[1]
    deg = acc1_ref[0, :, HID:HID + 1] + acc1_ref[1, :, HID:HID + 1]
    out_ref[...] = s2_ref[...] + agg / jnp.maximum(deg, 1.0) + b2_ref[...]


def _dis_call(d, m1, wd, bd, w1, ws1, wn1):
    blk = 512
    grid = (ND + blk - 1) // blk
    return pl.pallas_call(
        _dis_body,
        grid=(grid,),
        in_specs=[
            pl.BlockSpec((blk, 383), lambda i: (i, 0)),
            pl.BlockSpec((2000, blk), lambda i: (0, i)),
            pl.BlockSpec((383, EMB), lambda i: (0, 0)),
            pl.BlockSpec((1, EMB), lambda i: (0, 0)),
            pl.BlockSpec((2000, EMB), lambda i: (0, 0)),
            pl.BlockSpec((EMB, HID), lambda i: (0, 0)),
            pl.BlockSpec((EMB, HID), lambda i: (0, 0)),
        ],
        out_specs=[
            pl.BlockSpec((blk, HID), lambda i: (i, 0)),
            pl.BlockSpec((blk, TW1), lambda i: (i, 0)),
        ],
        out_shape=[
            jax.ShapeDtypeStruct((ND, HID), jnp.float32),
            jax.ShapeDtypeStruct((ND, TW1), jnp.float32),
        ],
    )(d, m1, wd, bd, w1, ws1, wn1)


def _gen_call(g, m2, wg, bg, w4, ws1, wn1):
    blk = 256
    grid = (NG + blk - 1) // blk
    return pl.pallas_call(
        _gen_body,
        grid=(grid,),
        in_specs=[
            pl.BlockSpec((blk, 4395), lambda i: (i, 0)),
            pl.BlockSpec((blk, 4000), lambda i: (i, 0)),
            pl.BlockSpec((4395, EMB), lambda i: (0, 0)),
            pl.BlockSpec((1, EMB), lambda i: (0, 0)),
            pl.BlockSpec((4000, EMB), lambda i: (0, 0)),
            pl.BlockSpec((EMB, HID), lambda i: (0, 0)),
            pl.BlockSpec((EMB, HID), lambda i: (0, 0)),
        ],
        out_specs=[
            pl.BlockSpec((blk, HID), lambda i: (i, 0)),
            pl.BlockSpec((blk, TW1), lambda i: (i, 0)),
        ],
        out_shape=[
            jax.ShapeDtypeStruct((NG, HID), jnp.float32),
            jax.ShapeDtypeStruct((NG, TW1), jnp.float32),
        ],
    )(g, m2, wg, bg, w4, ws1, wn1)


def _combine1_call(s1, acc, b1, wn2, ws2):
    blk = 512
    grid = (NN + blk - 1) // blk
    return pl.pallas_call(
        _combine1_body,
        grid=(grid,),
        in_specs=[
            pl.BlockSpec((blk, HID), lambda i: (i, 0)),
            pl.BlockSpec((NC, blk, TW1), lambda i: (0, i, 0)),
            pl.BlockSpec((1, HID), lambda i: (0, 0)),
            pl.BlockSpec((HID, HID), lambda i: (0, 0)),
            pl.BlockSpec((HID, HID), lambda i: (0, 0)),
        ],
        out_specs=[
            pl.BlockSpec((blk, HID), lambda i: (i, 0)),
            pl.BlockSpec((blk, HID), lambda i: (i, 0)),
        ],
        out_shape=[
            jax.ShapeDtypeStruct((NN, HID), jnp.float32),
            jax.ShapeDtypeStruct((NN, HID), jnp.float32),
        ],
    )(s1, acc, b1, wn2, ws2)


def _combine2_call(s2, acc2, acc1, b2):
    blk = 512
    grid = (NN + blk - 1) // blk
    return pl.pallas_call(
        _combine2_body,
        grid=(grid,),
        in_specs=[
            pl.BlockSpec((blk, HID), lambda i: (i, 0)),
            pl.BlockSpec((NC, blk, HID), lambda i: (0, i, 0)),
            pl.BlockSpec((NC, blk, TW1), lambda i: (0, i, 0)),
            pl.BlockSpec((1, HID), lambda i: (0, 0)),
        ],
        out_specs=pl.BlockSpec((blk, HID), lambda i: (i, 0)),
        out_shape=jax.ShapeDtypeStruct((NN, HID), jnp.float32),
    )(s2, acc2, acc1, b2)


def _sc_body(tw, nj, *refs):
    (t_hbm, src_hbm, dst_hbm, za_hbm,
     acc_out,
     idx_s, idx_d, rows, acc_sh, sem_g, sem_s) = refs
    nsup = OPS_PER_W // nj
    c = lax.axis_index("c")
    s = lax.axis_index("s")
    wid = s * NC + c
    opbase = wid * OPS_PER_W

    pltpu.sync_copy(za_hbm, acc_sh.at[pl.ds(s * RPT, RPT)])
    pltpu.sync_copy(src_hbm.at[pl.ds(opbase, OPS_PER_W)], idx_s)
    pltpu.sync_copy(dst_hbm.at[pl.ds(opbase, OPS_PER_W)], idx_d)
    plsc.subcore_barrier()

    def fire_gathers(u, b):
        for j in range(nj):
            pltpu.async_copy(t_hbm.at[idx_s.at[u * nj + j]],
                             rows.at[b, j], sem_g)

    def drain(sem):
        for _ in range(nj):
            pltpu.make_async_copy(t_hbm.at[pl.ds(0, SLICE)],
                                  rows.at[0, 0], sem).wait()

    fire_gathers(0, 0)

    def step(u, carry):
        b = lax.rem(u, 2)
        nb = 1 - b
        drain(sem_g)

        @pl.when(u >= 1)
        def _():
            drain(sem_s)

        @pl.when(u <= nsup - 2)
        def _():
            fire_gathers(u + 1, nb)

        for j in range(nj):
            pltpu.async_copy(rows.at[b, j], acc_sh.at[idx_d.at[u * nj + j]],
                             sem_s, add=True)
        return carry

    lax.fori_loop(0, nsup, step, 0)
    drain(sem_s)
    plsc.subcore_barrier()
    pltpu.sync_copy(acc_sh.at[pl.ds(s * RPT, RPT)],
                    acc_out.at[c, pl.ds(s * RPT, RPT)])


@functools.lru_cache(maxsize=None)
def _sc_agg_kernel(tw, nj):
    return functools.partial(
        pl.kernel,
        mesh=plsc.VectorSubcoreMesh(core_axis_name="c", subcore_axis_name="s"),
        compiler_params=pltpu.CompilerParams(use_tc_tiling_on_sc=False),
        out_type=jax.ShapeDtypeStruct((NC, R_PAD, tw), jnp.float32),
        scratch_types=[
            pltpu.VMEM((OPS_PER_W, SLICE), jnp.int32),
            pltpu.VMEM((OPS_PER_W, SLICE), jnp.int32),
            pltpu.VMEM((2, nj, SLICE, tw), jnp.float32),
            pltpu.VMEM_SHARED((R_PAD, tw), jnp.float32),
            pltpu.SemaphoreType.DMA,
            pltpu.SemaphoreType.DMA,
        ],
    )(functools.partial(_sc_body, tw, nj))


def kernel(d_features, g_features, M1_mirna_dis, M2_gene_dis, edge_index,
           W1, W2, W3, W4, Wd, bd, Wg, bg, Ws1, Wn1, b1, Ws2, Wn2, b2):
    src = edge_index[0]
    dst = edge_index[1]
    npad = NE_PAD - NE
    pad_ids = lax.iota(jnp.int32, npad)
    src_p = jnp.concatenate([src, pad_ids % NN])
    dst_p = jnp.concatenate([dst, NN + pad_ids % (R_PAD - NN)])
    src2 = src_p.reshape(NE_PAD // SLICE, SLICE)
    dst2 = dst_p.reshape(NE_PAD // SLICE, SLICE)
    za1 = jnp.zeros((RPT, TW1), jnp.float32)
    za2 = jnp.zeros((RPT, HID), jnp.float32)

    bd2 = bd.reshape(1, EMB)
    bg2 = bg.reshape(1, EMB)
    b12 = b1.reshape(1, HID)
    b22 = b2.reshape(1, HID)

    s1d, t1d = _dis_call(d_features, M1_mirna_dis, Wd, bd2, W1, Ws1, Wn1)
    s1g, t1g = _gen_call(g_features, M2_gene_dis, Wg, bg2, W4, Ws1, Wn1)
    s1 = jnp.concatenate([s1d, s1g], axis=0)
    t1 = jnp.concatenate([t1d, t1g], axis=0)

    acc1 = _sc_agg_kernel(TW1, 2)(t1, src2, dst2, za1)
    t2, s2 = _combine1_call(s1, acc1, b12, Wn2, Ws2)
    acc2 = _sc_agg_kernel(HID, 4)(t2, src2, dst2, za2)
    return _combine2_call(s2, acc2, acc1, b22)


# speedup vs baseline: 9.2921x; 1.4747x over previous
import functools

import jax
import jax.numpy as jnp
from jax import lax
from jax.experimental import pallas as pl
from jax.experimental.pallas import tpu as pltpu
from jax.experimental.pallas import tpu_sc as plsc

ND = 4000
NG = 6000
NN = ND + NG
NE = 320000
EMB = 128
HID = 64
TW1 = HID + 16

NC = 2
NS = 16
NW = NC * NS

SLICE = 128
OPS_PER_W = 80
EPW = OPS_PER_W * SLICE
NE_PAD = EPW * NW

R_PAD = 10240
RPT = R_PAD // NS


def _dis_body(d_ref, m1_ref, wd_ref, bd_ref, w1_ref, ws1_ref, wn1_ref,
              s1_ref, t1_ref):
    rep = 0.9 * (jnp.dot(d_ref[...], wd_ref[...],
                         preferred_element_type=jnp.float32) + bd_ref[...])
    rep = rep + 0.1 * lax.dot_general(
        m1_ref[...], w1_ref[...], (((0,), (0,)), ((), ())),
        preferred_element_type=jnp.float32)
    s1_ref[...] = jnp.dot(rep, ws1_ref[...], preferred_element_type=jnp.float32)
    t1 = jnp.dot(rep, wn1_ref[...], preferred_element_type=jnp.float32)
    t1_ref[...] = jnp.concatenate(
        [t1, jnp.ones((t1.shape[0], 16), jnp.float32)], axis=1)


def _gen_body(gt_ref, m2t_ref, wg_ref, bg_ref, w4_ref, ws1_ref, wn1_ref,
              s1_ref, t1_ref):
    rep = 0.9 * (lax.dot_general(
        gt_ref[...], wg_ref[...], (((0,), (0,)), ((), ())),
        preferred_element_type=jnp.float32) + bg_ref[...])
    rep = rep + 0.1 * lax.dot_general(
        m2t_ref[...], w4_ref[...], (((0,), (0,)), ((), ())),
        preferred_element_type=jnp.float32)
    s1_ref[...] = jnp.dot(rep, ws1_ref[...], preferred_element_type=jnp.float32)
    t1 = jnp.dot(rep, wn1_ref[...], preferred_element_type=jnp.float32)
    t1_ref[...] = jnp.concatenate(
        [t1, jnp.ones((t1.shape[0], 16), jnp.float32)], axis=1)


def _combine1_body(s1_ref, acc_ref, b1_ref, wn2_ref, ws2_ref,
                   t2_ref, s2_ref):
    a = acc_ref[0] + acc_ref[1]
    agg = a[:, :HID]
    deg = a[:, HID:HID + 1]
    hn = agg / jnp.maximum(deg, 1.0)
    h1 = jnp.maximum(s1_ref[...] + hn + b1_ref[...], 0.0)
    t2_ref[...] = jnp.dot(h1, wn2_ref[...], preferred_element_type=jnp.float32)
    s2_ref[...] = jnp.dot(h1, ws2_ref[...], preferred_element_type=jnp.float32)


def _combine2_body(s2_ref, acc_ref, acc1_ref, b2_ref, out_ref):
    agg = acc_ref[0] + acc_ref[1]
    deg = acc1_ref[0, :, HID:HID + 1] + acc1_ref[1, :, HID:HID + 1]
    out_ref[...] = s2_ref[...] + agg / jnp.maximum(deg, 1.0) + b2_ref[...]


def _dis_call(d, m1, wd, bd, w1, ws1, wn1):
    blk = 512
    grid = (ND + blk - 1) // blk
    return pl.pallas_call(
        _dis_body,
        grid=(grid,),
        in_specs=[
            pl.BlockSpec((blk, 383), lambda i: (i, 0)),
            pl.BlockSpec((2000, blk), lambda i: (0, i)),
            pl.BlockSpec((383, EMB), lambda i: (0, 0)),
            pl.BlockSpec((1, EMB), lambda i: (0, 0)),
            pl.BlockSpec((2000, EMB), lambda i: (0, 0)),
            pl.BlockSpec((EMB, HID), lambda i: (0, 0)),
            pl.BlockSpec((EMB, HID), lambda i: (0, 0)),
        ],
        out_specs=[
            pl.BlockSpec((blk, HID), lambda i: (i, 0)),
            pl.BlockSpec((blk, TW1), lambda i: (i, 0)),
        ],
        out_shape=[
            jax.ShapeDtypeStruct((ND, HID), jnp.float32),
            jax.ShapeDtypeStruct((ND, TW1), jnp.float32),
        ],
    )(d, m1, wd, bd, w1, ws1, wn1)


def _gen_call(gt, m2t, wg, bg, w4, ws1, wn1):
    blk = 256
    grid = (NG + blk - 1) // blk
    return pl.pallas_call(
        _gen_body,
        grid=(grid,),
        in_specs=[
            pl.BlockSpec((4395, blk), lambda i: (0, i)),
            pl.BlockSpec((4000, blk), lambda i: (0, i)),
            pl.BlockSpec((4395, EMB), lambda i: (0, 0)),
            pl.BlockSpec((1, EMB), lambda i: (0, 0)),
            pl.BlockSpec((4000, EMB), lambda i: (0, 0)),
            pl.BlockSpec((EMB, HID), lambda i: (0, 0)),
            pl.BlockSpec((EMB, HID), lambda i: (0, 0)),
        ],
        out_specs=[
            pl.BlockSpec((blk, HID), lambda i: (i, 0)),
            pl.BlockSpec((blk, TW1), lambda i: (i, 0)),
        ],
        out_shape=[
            jax.ShapeDtypeStruct((NG, HID), jnp.float32),
            jax.ShapeDtypeStruct((NG, TW1), jnp.float32),
        ],
    )(gt, m2t, wg, bg, w4, ws1, wn1)


def _combine1_call(s1, acc, b1, wn2, ws2):
    blk = 512
    grid = (NN + blk - 1) // blk
    return pl.pallas_call(
        _combine1_body,
        grid=(grid,),
        in_specs=[
            pl.BlockSpec((blk, HID), lambda i: (i, 0)),
            pl.BlockSpec((NC, blk, TW1), lambda i: (0, i, 0)),
            pl.BlockSpec((1, HID), lambda i: (0, 0)),
            pl.BlockSpec((HID, HID), lambda i: (0, 0)),
            pl.BlockSpec((HID, HID), lambda i: (0, 0)),
        ],
        out_specs=[
            pl.BlockSpec((blk, HID), lambda i: (i, 0)),
            pl.BlockSpec((blk, HID), lambda i: (i, 0)),
        ],
        out_shape=[
            jax.ShapeDtypeStruct((NN, HID), jnp.float32),
            jax.ShapeDtypeStruct((NN, HID), jnp.float32),
        ],
    )(s1, acc, b1, wn2, ws2)


def _combine2_call(s2, acc2, acc1, b2):
    blk = 512
    grid = (NN + blk - 1) // blk
    return pl.pallas_call(
        _combine2_body,
        grid=(grid,),
        in_specs=[
            pl.BlockSpec((blk, HID), lambda i: (i, 0)),
            pl.BlockSpec((NC, blk, HID), lambda i: (0, i, 0)),
            pl.BlockSpec((NC, blk, TW1), lambda i: (0, i, 0)),
            pl.BlockSpec((1, HID), lambda i: (0, 0)),
        ],
        out_specs=pl.BlockSpec((blk, HID), lambda i: (i, 0)),
        out_shape=jax.ShapeDtypeStruct((NN, HID), jnp.float32),
    )(s2, acc2, acc1, b2)


def _sc_body(tw, nj, *refs):
    (t_hbm, src_hbm, dst_hbm, za_hbm,
     acc_out,
     idx_s, idx_d, rows, acc_sh, sem_g, sem_s) = refs
    nsup = OPS_PER_W // nj
    c = lax.axis_index("c")
    s = lax.axis_index("s")
    wid = s * NC + c
    opbase = wid * OPS_PER_W

    pltpu.sync_copy(za_hbm, acc_sh.at[pl.ds(s * RPT, RPT)])
    pltpu.sync_copy(src_hbm.at[pl.ds(opbase, OPS_PER_W)], idx_s)
    pltpu.sync_copy(dst_hbm.at[pl.ds(opbase, OPS_PER_W)], idx_d)
    plsc.subcore_barrier()

    def fire_gathers(u, b):
        for j in range(nj):
            pltpu.async_copy(t_hbm.at[idx_s.at[u * nj + j]],
                             rows.at[b, j], sem_g)

    def drain(sem):
        for _ in range(nj):
            pltpu.make_async_copy(t_hbm.at[pl.ds(0, SLICE)],
                                  rows.at[0, 0], sem).wait()

    fire_gathers(0, 0)

    def step(u, carry):
        b = lax.rem(u, 2)
        nb = 1 - b
        drain(sem_g)

        @pl.when(u >= 1)
        def _():
            drain(sem_s)

        @pl.when(u <= nsup - 2)
        def _():
            fire_gathers(u + 1, nb)

        for j in range(nj):
            pltpu.async_copy(rows.at[b, j], acc_sh.at[idx_d.at[u * nj + j]],
                             sem_s, add=True)
        return carry

    lax.fori_loop(0, nsup, step, 0)
    drain(sem_s)
    plsc.subcore_barrier()
    pltpu.sync_copy(acc_sh.at[pl.ds(s * RPT, RPT)],
                    acc_out.at[c, pl.ds(s * RPT, RPT)])


@functools.lru_cache(maxsize=None)
def _sc_agg_kernel(tw, nj):
    return functools.partial(
        pl.kernel,
        mesh=plsc.VectorSubcoreMesh(core_axis_name="c", subcore_axis_name="s"),
        compiler_params=pltpu.CompilerParams(use_tc_tiling_on_sc=False),
        out_type=jax.ShapeDtypeStruct((NC, R_PAD, tw), jnp.float32),
        scratch_types=[
            pltpu.VMEM((OPS_PER_W, SLICE), jnp.int32),
            pltpu.VMEM((OPS_PER_W, SLICE), jnp.int32),
            pltpu.VMEM((2, nj, SLICE, tw), jnp.float32),
            pltpu.VMEM_SHARED((R_PAD, tw), jnp.float32),
            pltpu.SemaphoreType.DMA,
            pltpu.SemaphoreType.DMA,
        ],
    )(functools.partial(_sc_body, tw, nj))


def kernel(d_features, g_features, M1_mirna_dis, M2_gene_dis, edge_index,
           W1, W2, W3, W4, Wd, bd, Wg, bg, Ws1, Wn1, b1, Ws2, Wn2, b2):
    src = edge_index[0]
    dst = edge_index[1]
    npad = NE_PAD - NE
    pad_ids = lax.iota(jnp.int32, npad)
    src_p = jnp.concatenate([src, pad_ids % NN])
    dst_p = jnp.concatenate([dst, NN + pad_ids % (R_PAD - NN)])
    src2 = src_p.reshape(NE_PAD // SLICE, SLICE)
    dst2 = dst_p.reshape(NE_PAD // SLICE, SLICE)
    za1 = jnp.zeros((RPT, TW1), jnp.float32)
    za2 = jnp.zeros((RPT, HID), jnp.float32)

    bd2 = bd.reshape(1, EMB)
    bg2 = bg.reshape(1, EMB)
    b12 = b1.reshape(1, HID)
    b22 = b2.reshape(1, HID)

    s1d, t1d = _dis_call(d_features, M1_mirna_dis, Wd, bd2, W1, Ws1, Wn1)
    s1g, t1g = _gen_call(g_features.T, M2_gene_dis.T, Wg, bg2, W4, Ws1, Wn1)
    s1 = jnp.concatenate([s1d, s1g], axis=0)
    t1 = jnp.concatenate([t1d, t1g], axis=0)

    acc1 = _sc_agg_kernel(TW1, 2)(t1, src2, dst2, za1)
    t2, s2 = _combine1_call(s1, acc1, b12, Wn2, Ws2)
    acc2 = _sc_agg_kernel(HID, 4)(t2, src2, dst2, za2)
    return _combine2_call(s2, acc2, acc1, b22)


# speedup vs baseline: 9.9407x; 1.0698x over previous
import functools

import jax
import jax.numpy as jnp
from jax import lax
from jax.experimental import pallas as pl
from jax.experimental.pallas import tpu as pltpu
from jax.experimental.pallas import tpu_sc as plsc

ND = 4000
NG = 6000
NN = ND + NG
NE = 320000
EMB = 128
HID = 64
TW1 = HID + 16

NC = 2
NS = 16
NW = NC * NS

SLICE = 128
OPS_PER_W = 80
EPW = OPS_PER_W * SLICE
NE_PAD = EPW * NW

R_PAD = 10240
RPT = R_PAD // NS


def _dis_body(d_ref, m1_ref, wd_ref, bd_ref, w1_ref, ws1_ref, wn1_ref,
              s1_ref, t1_ref):
    rep = 0.9 * (jnp.dot(d_ref[...], wd_ref[...],
                         preferred_element_type=jnp.float32) + bd_ref[...])
    rep = rep + 0.1 * lax.dot_general(
        m1_ref[...], w1_ref[...], (((0,), (0,)), ((), ())),
        preferred_element_type=jnp.float32)
    s1_ref[...] = jnp.dot(rep, ws1_ref[...], preferred_element_type=jnp.float32)
    t1 = jnp.dot(rep, wn1_ref[...], preferred_element_type=jnp.float32)
    t1_ref[...] = jnp.concatenate(
        [t1, jnp.ones((t1.shape[0], 16), jnp.float32)], axis=1)


def _gen_body(gt_ref, m2t_ref, wg_ref, bg_ref, w4_ref, ws1_ref, wn1_ref,
              s1_ref, t1_ref):
    rep = 0.9 * (lax.dot_general(
        gt_ref[...], wg_ref[...], (((0,), (0,)), ((), ())),
        preferred_element_type=jnp.float32) + bg_ref[...])
    rep = rep + 0.1 * lax.dot_general(
        m2t_ref[...], w4_ref[...], (((0,), (0,)), ((), ())),
        preferred_element_type=jnp.float32)
    s1_ref[...] = jnp.dot(rep, ws1_ref[...], preferred_element_type=jnp.float32)
    t1 = jnp.dot(rep, wn1_ref[...], preferred_element_type=jnp.float32)
    t1_ref[...] = jnp.concatenate(
        [t1, jnp.ones((t1.shape[0], 16), jnp.float32)], axis=1)


def _combine1_body(s1_ref, acc_ref, b1_ref, wn2_ref, ws2_ref,
                   t2_ref, s2_ref):
    a = acc_ref[0] + acc_ref[1]
    agg = a[:, :HID]
    deg = a[:, HID:HID + 1]
    hn = agg / jnp.maximum(deg, 1.0)
    h1 = jnp.maximum(s1_ref[...] + hn + b1_ref[...], 0.0)
    t2_ref[...] = jnp.dot(h1, wn2_ref[...], preferred_element_type=jnp.float32)
    s2_ref[...] = jnp.dot(h1, ws2_ref[...], preferred_element_type=jnp.float32)


def _combine2_body(s2_ref, acc_ref, acc1_ref, b2_ref, out_ref):
    agg = acc_ref[0] + acc_ref[1]
    deg = acc1_ref[0, :, HID:HID + 1] + acc1_ref[1, :, HID:HID + 1]
    out_ref[...] = s2_ref[...] + agg / jnp.maximum(deg, 1.0) + b2_ref[...]


def _dis_call(d, m1, wd, bd, w1, ws1, wn1):
    blk = 1024
    grid = (ND + blk - 1) // blk
    return pl.pallas_call(
        _dis_body,
        grid=(grid,),
        in_specs=[
            pl.BlockSpec((blk, 383), lambda i: (i, 0)),
            pl.BlockSpec((2000, blk), lambda i: (0, i)),
            pl.BlockSpec((383, EMB), lambda i: (0, 0)),
            pl.BlockSpec((1, EMB), lambda i: (0, 0)),
            pl.BlockSpec((2000, EMB), lambda i: (0, 0)),
            pl.BlockSpec((EMB, HID), lambda i: (0, 0)),
            pl.BlockSpec((EMB, HID), lambda i: (0, 0)),
        ],
        out_specs=[
            pl.BlockSpec((blk, HID), lambda i: (i, 0)),
            pl.BlockSpec((blk, TW1), lambda i: (i, 0)),
        ],
        out_shape=[
            jax.ShapeDtypeStruct((ND, HID), jnp.float32),
            jax.ShapeDtypeStruct((ND, TW1), jnp.float32),
        ],
    )(d, m1, wd, bd, w1, ws1, wn1)


def _gen_call(gt, m2t, wg, bg, w4, ws1, wn1):
    blk = 512
    grid = (NG + blk - 1) // blk
    return pl.pallas_call(
        _gen_body,
        grid=(grid,),
        in_specs=[
            pl.BlockSpec((4395, blk), lambda i: (0, i)),
            pl.BlockSpec((4000, blk), lambda i: (0, i)),
            pl.BlockSpec((4395, EMB), lambda i: (0, 0)),
            pl.BlockSpec((1, EMB), lambda i: (0, 0)),
            pl.BlockSpec((4000, EMB), lambda i: (0, 0)),
            pl.BlockSpec((EMB, HID), lambda i: (0, 0)),
            pl.BlockSpec((EMB, HID), lambda i: (0, 0)),
        ],
        out_specs=[
            pl.BlockSpec((blk, HID), lambda i: (i, 0)),
            pl.BlockSpec((blk, TW1), lambda i: (i, 0)),
        ],
        out_shape=[
            jax.ShapeDtypeStruct((NG, HID), jnp.float32),
            jax.ShapeDtypeStruct((NG, TW1), jnp.float32),
        ],
    )(gt, m2t, wg, bg, w4, ws1, wn1)


def _combine1_call(s1, acc, b1, wn2, ws2):
    blk = 2048
    grid = (NN + blk - 1) // blk
    return pl.pallas_call(
        _combine1_body,
        grid=(grid,),
        in_specs=[
            pl.BlockSpec((blk, HID), lambda i: (i, 0)),
            pl.BlockSpec((NC, blk, TW1), lambda i: (0, i, 0)),
            pl.BlockSpec((1, HID), lambda i: (0, 0)),
            pl.BlockSpec((HID, HID), lambda i: (0, 0)),
            pl.BlockSpec((HID, HID), lambda i: (0, 0)),
        ],
        out_specs=[
            pl.BlockSpec((blk, HID), lambda i: (i, 0)),
            pl.BlockSpec((blk, HID), lambda i: (i, 0)),
        ],
        out_shape=[
            jax.ShapeDtypeStruct((NN, HID), jnp.float32),
            jax.ShapeDtypeStruct((NN, HID), jnp.float32),
        ],
    )(s1, acc, b1, wn2, ws2)


def _combine2_call(s2, acc2, acc1, b2):
    blk = 2048
    grid = (NN + blk - 1) // blk
    return pl.pallas_call(
        _combine2_body,
        grid=(grid,),
        in_specs=[
            pl.BlockSpec((blk, HID), lambda i: (i, 0)),
            pl.BlockSpec((NC, blk, HID), lambda i: (0, i, 0)),
            pl.BlockSpec((NC, blk, TW1), lambda i: (0, i, 0)),
            pl.BlockSpec((1, HID), lambda i: (0, 0)),
        ],
        out_specs=pl.BlockSpec((blk, HID), lambda i: (i, 0)),
        out_shape=jax.ShapeDtypeStruct((NN, HID), jnp.float32),
    )(s2, acc2, acc1, b2)


def _sc_body(tw, nj, *refs):
    (t_hbm, src_hbm, dst_hbm, za_hbm,
     acc_out,
     idx_s, idx_d, rows, acc_sh, sem_g, sem_s) = refs
    nsup = OPS_PER_W // nj
    c = lax.axis_index("c")
    s = lax.axis_index("s")
    wid = s * NC + c
    opbase = wid * OPS_PER_W

    pltpu.sync_copy(za_hbm, acc_sh.at[pl.ds(s * RPT, RPT)])
    pltpu.sync_copy(src_hbm.at[pl.ds(opbase, OPS_PER_W)], idx_s)
    pltpu.sync_copy(dst_hbm.at[pl.ds(opbase, OPS_PER_W)], idx_d)
    plsc.subcore_barrier()

    def fire_gathers(u, b):
        for j in range(nj):
            pltpu.async_copy(t_hbm.at[idx_s.at[u * nj + j]],
                             rows.at[b, j], sem_g)

    def drain(sem):
        for _ in range(nj):
            pltpu.make_async_copy(t_hbm.at[pl.ds(0, SLICE)],
                                  rows.at[0, 0], sem).wait()

    fire_gathers(0, 0)

    def step(u, carry):
        b = lax.rem(u, 2)
        nb = 1 - b
        drain(sem_g)

        @pl.when(u >= 1)
        def _():
            drain(sem_s)

        @pl.when(u <= nsup - 2)
        def _():
            fire_gathers(u + 1, nb)

        for j in range(nj):
            pltpu.async_copy(rows.at[b, j], acc_sh.at[idx_d.at[u * nj + j]],
                             sem_s, add=True)
        return carry

    lax.fori_loop(0, nsup, step, 0)
    drain(sem_s)
    plsc.subcore_barrier()
    pltpu.sync_copy(acc_sh.at[pl.ds(s * RPT, RPT)],
                    acc_out.at[c, pl.ds(s * RPT, RPT)])


@functools.lru_cache(maxsize=None)
def _sc_agg_kernel(tw, nj):
    return functools.partial(
        pl.kernel,
        mesh=plsc.VectorSubcoreMesh(core_axis_name="c", subcore_axis_name="s"),
        compiler_params=pltpu.CompilerParams(use_tc_tiling_on_sc=False),
        out_type=jax.ShapeDtypeStruct((NC, R_PAD, tw), jnp.float32),
        scratch_types=[
            pltpu.VMEM((OPS_PER_W, SLICE), jnp.int32),
            pltpu.VMEM((OPS_PER_W, SLICE), jnp.int32),
            pltpu.VMEM((2, nj, SLICE, tw), jnp.float32),
            pltpu.VMEM_SHARED((R_PAD, tw), jnp.float32),
            pltpu.SemaphoreType.DMA,
            pltpu.SemaphoreType.DMA,
        ],
    )(functools.partial(_sc_body, tw, nj))


def kernel(d_features, g_features, M1_mirna_dis, M2_gene_dis, edge_index,
           W1, W2, W3, W4, Wd, bd, Wg, bg, Ws1, Wn1, b1, Ws2, Wn2, b2):
    src = edge_index[0]
    dst = edge_index[1]
    npad = NE_PAD - NE
    pad_ids = lax.iota(jnp.int32, npad)
    src_p = jnp.concatenate([src, pad_ids % NN])
    dst_p = jnp.concatenate([dst, NN + pad_ids % (R_PAD - NN)])
    src2 = src_p.reshape(NE_PAD // SLICE, SLICE)
    dst2 = dst_p.reshape(NE_PAD // SLICE, SLICE)
    za1 = jnp.zeros((RPT, TW1), jnp.float32)
    za2 = jnp.zeros((RPT, HID), jnp.float32)

    bd2 = bd.reshape(1, EMB)
    bg2 = bg.reshape(1, EMB)
    b12 = b1.reshape(1, HID)
    b22 = b2.reshape(1, HID)

    s1d, t1d = _dis_call(d_features, M1_mirna_dis, Wd, bd2, W1, Ws1, Wn1)
    s1g, t1g = _gen_call(g_features.T, M2_gene_dis.T, Wg, bg2, W4, Ws1, Wn1)
    s1 = jnp.concatenate([s1d, s1g], axis=0)
    t1 = jnp.concatenate([t1d, t1g], axis=0)

    acc1 = _sc_agg_kernel(TW1, 2)(t1, src2, dst2, za1)
    t2, s2 = _combine1_call(s1, acc1, b12, Wn2, Ws2)
    acc2 = _sc_agg_kernel(HID, 4)(t2, src2, dst2, za2)
    return _combine2_call(s2, acc2, acc1, b22)


# speedup vs baseline: 11.3612x; 1.1429x over previous
import functools

import jax
import jax.numpy as jnp
from jax import lax
from jax.experimental import pallas as pl
from jax.experimental.pallas import tpu as pltpu
from jax.experimental.pallas import tpu_sc as plsc

ND = 4000
NG = 6000
NN = ND + NG
NE = 320000
EMB = 128
HID = 64
TW1 = HID + 16

NC = 2
NS = 16
NW = NC * NS

SLICE = 128
OPS_PER_W = 80
EPW = OPS_PER_W * SLICE
NE_PAD = EPW * NW

R_PAD = 10240
RPT = R_PAD // NS


def _dis_body(d_ref, m1_ref, wd_ref, bd_ref, w1_ref, ws1_ref, wn1_ref,
              s1_ref, t1_ref):
    rep = 0.9 * (jnp.dot(d_ref[...], wd_ref[...],
                         preferred_element_type=jnp.float32) + bd_ref[...])
    rep = rep + 0.1 * lax.dot_general(
        m1_ref[...], w1_ref[...], (((0,), (0,)), ((), ())),
        preferred_element_type=jnp.float32)
    s1_ref[...] = jnp.dot(rep, ws1_ref[...], preferred_element_type=jnp.float32)
    t1 = jnp.dot(rep, wn1_ref[...], preferred_element_type=jnp.float32)
    t1_ref[...] = jnp.concatenate(
        [t1, jnp.ones((t1.shape[0], 16), jnp.float32)], axis=1)


def _gen_body(gt_ref, m2t_ref, wg_ref, bg_ref, w4_ref, ws1_ref, wn1_ref,
              s1_ref, t1_ref):
    rep = 0.9 * (lax.dot_general(
        gt_ref[...], wg_ref[...], (((0,), (0,)), ((), ())),
        preferred_element_type=jnp.float32) + bg_ref[...])
    rep = rep + 0.1 * lax.dot_general(
        m2t_ref[...], w4_ref[...], (((0,), (0,)), ((), ())),
        preferred_element_type=jnp.float32)
    s1_ref[...] = jnp.dot(rep, ws1_ref[...], preferred_element_type=jnp.float32)
    t1 = jnp.dot(rep, wn1_ref[...], preferred_element_type=jnp.float32)
    t1_ref[...] = jnp.concatenate(
        [t1, jnp.ones((t1.shape[0], 16), jnp.float32)], axis=1)


def _combine1_body(s1_ref, acc_ref, b1_ref, wn2_ref, ws2_ref,
                   t2_ref, s2_ref):
    a = acc_ref[0] + acc_ref[1]
    agg = a[:, :HID]
    deg = a[:, HID:HID + 1]
    hn = agg / jnp.maximum(deg, 1.0)
    h1 = jnp.maximum(s1_ref[...] + hn + b1_ref[...], 0.0)
    t2_ref[...] = jnp.dot(h1, wn2_ref[...], preferred_element_type=jnp.float32)
    s2_ref[...] = jnp.dot(h1, ws2_ref[...], preferred_element_type=jnp.float32)


def _combine2_body(s2_ref, acc_ref, acc1_ref, b2_ref, out_ref):
    agg = acc_ref[0] + acc_ref[1]
    deg = acc1_ref[0, :, HID:HID + 1] + acc1_ref[1, :, HID:HID + 1]
    out_ref[...] = s2_ref[...] + agg / jnp.maximum(deg, 1.0) + b2_ref[...]


def _dis_call(d, m1, wd, bd, w1, ws1, wn1):
    blk = 1024
    grid = (ND + blk - 1) // blk
    return pl.pallas_call(
        _dis_body,
        grid=(grid,),
        in_specs=[
            pl.BlockSpec((blk, 383), lambda i: (i, 0)),
            pl.BlockSpec((2000, blk), lambda i: (0, i)),
            pl.BlockSpec((383, EMB), lambda i: (0, 0)),
            pl.BlockSpec((1, EMB), lambda i: (0, 0)),
            pl.BlockSpec((2000, EMB), lambda i: (0, 0)),
            pl.BlockSpec((EMB, HID), lambda i: (0, 0)),
            pl.BlockSpec((EMB, HID), lambda i: (0, 0)),
        ],
        out_specs=[
            pl.BlockSpec((blk, HID), lambda i: (i, 0)),
            pl.BlockSpec((blk, TW1), lambda i: (i, 0)),
        ],
        out_shape=[
            jax.ShapeDtypeStruct((ND, HID), jnp.float32),
            jax.ShapeDtypeStruct((ND, TW1), jnp.float32),
        ],
    )(d, m1, wd, bd, w1, ws1, wn1)


def _gen_call(gt, m2t, wg, bg, w4, ws1, wn1):
    blk = 512
    grid = (NG + blk - 1) // blk
    return pl.pallas_call(
        _gen_body,
        grid=(grid,),
        in_specs=[
            pl.BlockSpec((4395, blk), lambda i: (0, i)),
            pl.BlockSpec((4000, blk), lambda i: (0, i)),
            pl.BlockSpec((4395, EMB), lambda i: (0, 0)),
            pl.BlockSpec((1, EMB), lambda i: (0, 0)),
            pl.BlockSpec((4000, EMB), lambda i: (0, 0)),
            pl.BlockSpec((EMB, HID), lambda i: (0, 0)),
            pl.BlockSpec((EMB, HID), lambda i: (0, 0)),
        ],
        out_specs=[
            pl.BlockSpec((blk, HID), lambda i: (i, 0)),
            pl.BlockSpec((blk, TW1), lambda i: (i, 0)),
        ],
        out_shape=[
            jax.ShapeDtypeStruct((NG, HID), jnp.float32),
            jax.ShapeDtypeStruct((NG, TW1), jnp.float32),
        ],
    )(gt, m2t, wg, bg, w4, ws1, wn1)


def _combine1_call(s1, acc, b1, wn2, ws2):
    blk = 2048
    grid = (NN + blk - 1) // blk
    return pl.pallas_call(
        _combine1_body,
        grid=(grid,),
        in_specs=[
            pl.BlockSpec((blk, HID), lambda i: (i, 0)),
            pl.BlockSpec((NC, blk, TW1), lambda i: (0, i, 0)),
            pl.BlockSpec((1, HID), lambda i: (0, 0)),
            pl.BlockSpec((HID, HID), lambda i: (0, 0)),
            pl.BlockSpec((HID, HID), lambda i: (0, 0)),
        ],
        out_specs=[
            pl.BlockSpec((blk, HID), lambda i: (i, 0)),
            pl.BlockSpec((blk, HID), lambda i: (i, 0)),
        ],
        out_shape=[
            jax.ShapeDtypeStruct((NN, HID), jnp.float32),
            jax.ShapeDtypeStruct((NN, HID), jnp.float32),
        ],
    )(s1, acc, b1, wn2, ws2)


def _combine2_call(s2, acc2, acc1, b2):
    blk = 2048
    grid = (NN + blk - 1) // blk
    return pl.pallas_call(
        _combine2_body,
        grid=(grid,),
        in_specs=[
            pl.BlockSpec((blk, HID), lambda i: (i, 0)),
            pl.BlockSpec((NC, blk, HID), lambda i: (0, i, 0)),
            pl.BlockSpec((NC, blk, TW1), lambda i: (0, i, 0)),
            pl.BlockSpec((1, HID), lambda i: (0, 0)),
        ],
        out_specs=pl.BlockSpec((blk, HID), lambda i: (i, 0)),
        out_shape=jax.ShapeDtypeStruct((NN, HID), jnp.float32),
    )(s2, acc2, acc1, b2)


NSLICE = NE // SLICE
MAXOPS = 79


def _sc_body(tw, gg, ss, *refs):
    (t_hbm, e_hbm, za_hbm,
     acc_out,
     idx_s, idx_d, rows, acc_sh, sem_g, sem_s) = refs
    depth = gg + ss
    c = lax.axis_index("c")
    s = lax.axis_index("s")
    wid = s * NC + c
    lo = wid * NSLICE // NW
    n = (wid + 1) * NSLICE // NW - lo

    pltpu.sync_copy(za_hbm, acc_sh.at[pl.ds(s * RPT, RPT)])
    pltpu.sync_copy(e_hbm.at[0, pl.ds(lo, MAXOPS)], idx_s)
    pltpu.sync_copy(e_hbm.at[1, pl.ds(lo, MAXOPS)], idx_d)
    plsc.subcore_barrier()

    def gather(u):
        pltpu.async_copy(t_hbm.at[idx_s.at[u]], rows.at[lax.rem(u, depth)],
                         sem_g)

    def drain(sem):
        pltpu.make_async_copy(t_hbm.at[pl.ds(0, SLICE)], rows.at[0],
                              sem).wait()

    for j in range(gg):
        gather(j)

    def step(u, carry):
        @pl.when(u >= ss)
        def _():
            drain(sem_s)

        @pl.when(u + gg <= n - 1)
        def _():
            gather(u + gg)

        drain(sem_g)
        pltpu.async_copy(rows.at[lax.rem(u, depth)],
                         acc_sh.at[idx_d.at[u]], sem_s, add=True)
        return carry

    lax.fori_loop(0, n, step, 0)
    for _ in range(ss):
        drain(sem_s)
    plsc.subcore_barrier()
    pltpu.sync_copy(acc_sh.at[pl.ds(s * RPT, RPT)],
                    acc_out.at[c, pl.ds(s * RPT, RPT)])


@functools.lru_cache(maxsize=None)
def _sc_agg_kernel(tw, gg, ss):
    return functools.partial(
        pl.kernel,
        mesh=plsc.VectorSubcoreMesh(core_axis_name="c", subcore_axis_name="s"),
        compiler_params=pltpu.CompilerParams(use_tc_tiling_on_sc=False),
        out_type=jax.ShapeDtypeStruct((NC, R_PAD, tw), jnp.float32),
        scratch_types=[
            pltpu.VMEM((MAXOPS, SLICE), jnp.int32),
            pltpu.VMEM((MAXOPS, SLICE), jnp.int32),
            pltpu.VMEM((gg + ss, SLICE, tw), jnp.float32),
            pltpu.VMEM_SHARED((R_PAD, tw), jnp.float32),
            pltpu.SemaphoreType.DMA,
            pltpu.SemaphoreType.DMA,
        ],
    )(functools.partial(_sc_body, tw, gg, ss))


def kernel(d_features, g_features, M1_mirna_dis, M2_gene_dis, edge_index,
           W1, W2, W3, W4, Wd, bd, Wg, bg, Ws1, Wn1, b1, Ws2, Wn2, b2):
    e3 = edge_index.reshape(2, NSLICE, SLICE)
    za1 = jnp.zeros((RPT, TW1), jnp.float32)
    za2 = jnp.zeros((RPT, HID), jnp.float32)

    bd2 = bd.reshape(1, EMB)
    bg2 = bg.reshape(1, EMB)
    b12 = b1.reshape(1, HID)
    b22 = b2.reshape(1, HID)

    s1d, t1d = _dis_call(d_features, M1_mirna_dis, Wd, bd2, W1, Ws1, Wn1)
    s1g, t1g = _gen_call(g_features.T, M2_gene_dis.T, Wg, bg2, W4, Ws1, Wn1)
    s1 = jnp.concatenate([s1d, s1g], axis=0)
    t1 = jnp.concatenate([t1d, t1g], axis=0)

    acc1 = _sc_agg_kernel(TW1, 2, 2)(t1, e3, za1)
    t2, s2 = _combine1_call(s1, acc1, b12, Wn2, Ws2)
    acc2 = _sc_agg_kernel(HID, 3, 3)(t2, e3, za2)
    return _combine2_call(s2, acc2, acc1, b22)


# speedup vs baseline: 11.4421x; 1.0071x over previous
import functools

import jax
import jax.numpy as jnp
from jax import lax
from jax.experimental import pallas as pl
from jax.experimental.pallas import tpu as pltpu
from jax.experimental.pallas import tpu_sc as plsc

ND = 4000
NG = 6000
NN = ND + NG
NE = 320000
EMB = 128
HID = 64
TW1 = HID + 16

NC = 2
NS = 16
NW = NC * NS

SLICE = 128
OPS_PER_W = 80
EPW = OPS_PER_W * SLICE
NE_PAD = EPW * NW

R_PAD = 10240
RPT = R_PAD // NS


def _dis_body(d_ref, m1_ref, wd_ref, bd_ref, w1_ref, ws1_ref, wn1_ref,
              s1_ref, t1_ref):
    rep = 0.9 * (jnp.dot(d_ref[...], wd_ref[...],
                         preferred_element_type=jnp.float32) + bd_ref[...])
    rep = rep + 0.1 * lax.dot_general(
        m1_ref[...], w1_ref[...], (((0,), (0,)), ((), ())),
        preferred_element_type=jnp.float32)
    s1_ref[...] = jnp.dot(rep, ws1_ref[...], preferred_element_type=jnp.float32)
    t1 = jnp.dot(rep, wn1_ref[...], preferred_element_type=jnp.float32)
    t1_ref[...] = jnp.concatenate(
        [t1, jnp.ones((t1.shape[0], 16), jnp.float32)], axis=1)


def _gen_body(gt_ref, m2t_ref, wg_ref, bg_ref, w4_ref, ws1_ref, wn1_ref,
              s1_ref, t1_ref):
    rep = 0.9 * (lax.dot_general(
        gt_ref[...], wg_ref[...], (((0,), (0,)), ((), ())),
        preferred_element_type=jnp.float32) + bg_ref[...])
    rep = rep + 0.1 * lax.dot_general(
        m2t_ref[...], w4_ref[...], (((0,), (0,)), ((), ())),
        preferred_element_type=jnp.float32)
    s1_ref[...] = jnp.dot(rep, ws1_ref[...], preferred_element_type=jnp.float32)
    t1 = jnp.dot(rep, wn1_ref[...], preferred_element_type=jnp.float32)
    t1_ref[...] = jnp.concatenate(
        [t1, jnp.ones((t1.shape[0], 16), jnp.float32)], axis=1)


def _combine1_body(s1_ref, acc_ref, b1_ref, wn2_ref, ws2_ref,
                   t2_ref, s2_ref):
    a = acc_ref[0] + acc_ref[1]
    agg = a[:, :HID]
    deg = a[:, HID:HID + 1]
    hn = agg / jnp.maximum(deg, 1.0)
    h1 = jnp.maximum(s1_ref[...] + hn + b1_ref[...], 0.0)
    t2_ref[...] = jnp.dot(h1, wn2_ref[...], preferred_element_type=jnp.float32)
    s2_ref[...] = jnp.dot(h1, ws2_ref[...], preferred_element_type=jnp.float32)


def _combine2_body(s2_ref, acc_ref, acc1_ref, b2_ref, out_ref):
    agg = acc_ref[0] + acc_ref[1]
    deg = acc1_ref[0, :, HID:HID + 1] + acc1_ref[1, :, HID:HID + 1]
    out_ref[...] = s2_ref[...] + agg / jnp.maximum(deg, 1.0) + b2_ref[...]


def _dis_call(d, m1, wd, bd, w1, ws1, wn1):
    blk = 2048
    grid = (ND + blk - 1) // blk
    return pl.pallas_call(
        _dis_body,
        grid=(grid,),
        in_specs=[
            pl.BlockSpec((blk, 383), lambda i: (i, 0)),
            pl.BlockSpec((2000, blk), lambda i: (0, i)),
            pl.BlockSpec((383, EMB), lambda i: (0, 0)),
            pl.BlockSpec((1, EMB), lambda i: (0, 0)),
            pl.BlockSpec((2000, EMB), lambda i: (0, 0)),
            pl.BlockSpec((EMB, HID), lambda i: (0, 0)),
            pl.BlockSpec((EMB, HID), lambda i: (0, 0)),
        ],
        out_specs=[
            pl.BlockSpec((blk, HID), lambda i: (i, 0)),
            pl.BlockSpec((blk, TW1), lambda i: (i, 0)),
        ],
        out_shape=[
            jax.ShapeDtypeStruct((ND, HID), jnp.float32),
            jax.ShapeDtypeStruct((ND, TW1), jnp.float32),
        ],
    )(d, m1, wd, bd, w1, ws1, wn1)


def _gen_call(gt, m2t, wg, bg, w4, ws1, wn1):
    blk = 512
    grid = (NG + blk - 1) // blk
    return pl.pallas_call(
        _gen_body,
        grid=(grid,),
        in_specs=[
            pl.BlockSpec((4395, blk), lambda i: (0, i)),
            pl.BlockSpec((4000, blk), lambda i: (0, i)),
            pl.BlockSpec((4395, EMB), lambda i: (0, 0)),
            pl.BlockSpec((1, EMB), lambda i: (0, 0)),
            pl.BlockSpec((4000, EMB), lambda i: (0, 0)),
            pl.BlockSpec((EMB, HID), lambda i: (0, 0)),
            pl.BlockSpec((EMB, HID), lambda i: (0, 0)),
        ],
        out_specs=[
            pl.BlockSpec((blk, HID), lambda i: (i, 0)),
            pl.BlockSpec((blk, TW1), lambda i: (i, 0)),
        ],
        out_shape=[
            jax.ShapeDtypeStruct((NG, HID), jnp.float32),
            jax.ShapeDtypeStruct((NG, TW1), jnp.float32),
        ],
    )(gt, m2t, wg, bg, w4, ws1, wn1)


def _combine1_call(s1, acc, b1, wn2, ws2):
    blk = 2560
    grid = (NN + blk - 1) // blk
    return pl.pallas_call(
        _combine1_body,
        grid=(grid,),
        in_specs=[
            pl.BlockSpec((blk, HID), lambda i: (i, 0)),
            pl.BlockSpec((NC, blk, TW1), lambda i: (0, i, 0)),
            pl.BlockSpec((1, HID), lambda i: (0, 0)),
            pl.BlockSpec((HID, HID), lambda i: (0, 0)),
            pl.BlockSpec((HID, HID), lambda i: (0, 0)),
        ],
        out_specs=[
            pl.BlockSpec((blk, HID), lambda i: (i, 0)),
            pl.BlockSpec((blk, HID), lambda i: (i, 0)),
        ],
        out_shape=[
            jax.ShapeDtypeStruct((NN, HID), jnp.float32),
            jax.ShapeDtypeStruct((NN, HID), jnp.float32),
        ],
    )(s1, acc, b1, wn2, ws2)


def _combine2_call(s2, acc2, acc1, b2):
    blk = 2560
    grid = (NN + blk - 1) // blk
    return pl.pallas_call(
        _combine2_body,
        grid=(grid,),
        in_specs=[
            pl.BlockSpec((blk, HID), lambda i: (i, 0)),
            pl.BlockSpec((NC, blk, HID), lambda i: (0, i, 0)),
            pl.BlockSpec((NC, blk, TW1), lambda i: (0, i, 0)),
            pl.BlockSpec((1, HID), lambda i: (0, 0)),
        ],
        out_specs=pl.BlockSpec((blk, HID), lambda i: (i, 0)),
        out_shape=jax.ShapeDtypeStruct((NN, HID), jnp.float32),
    )(s2, acc2, acc1, b2)


NSLICE = NE // SLICE
MAXOPS = 79


def _sc_body(tw, gg, ss, *refs):
    (t_hbm, e_hbm, za_hbm,
     acc_out,
     idx_s, idx_d, rows, acc_sh, sem_g, sem_s) = refs
    depth = gg + ss
    c = lax.axis_index("c")
    s = lax.axis_index("s")
    wid = s * NC + c
    lo = wid * NSLICE // NW
    n = (wid + 1) * NSLICE // NW - lo

    pltpu.sync_copy(za_hbm, acc_sh.at[pl.ds(s * RPT, RPT)])
    pltpu.sync_copy(e_hbm.at[0, pl.ds(lo, MAXOPS)], idx_s)
    pltpu.sync_copy(e_hbm.at[1, pl.ds(lo, MAXOPS)], idx_d)
    plsc.subcore_barrier()

    def gather(u):
        pltpu.async_copy(t_hbm.at[idx_s.at[u]], rows.at[lax.rem(u, depth)],
                         sem_g)

    def drain(sem):
        pltpu.make_async_copy(t_hbm.at[pl.ds(0, SLICE)], rows.at[0],
                              sem).wait()

    for j in range(gg):
        gather(j)

    def step(u, carry):
        @pl.when(u >= ss)
        def _():
            drain(sem_s)

        @pl.when(u + gg <= n - 1)
        def _():
            gather(u + gg)

        drain(sem_g)
        pltpu.async_copy(rows.at[lax.rem(u, depth)],
                         acc_sh.at[idx_d.at[u]], sem_s, add=True)
        return carry

    lax.fori_loop(0, n, step, 0)
    for _ in range(ss):
        drain(sem_s)
    plsc.subcore_barrier()
    pltpu.sync_copy(acc_sh.at[pl.ds(s * RPT, RPT)],
                    acc_out.at[c, pl.ds(s * RPT, RPT)])


@functools.lru_cache(maxsize=None)
def _sc_agg_kernel(tw, gg, ss):
    return functools.partial(
        pl.kernel,
        mesh=plsc.VectorSubcoreMesh(core_axis_name="c", subcore_axis_name="s"),
        compiler_params=pltpu.CompilerParams(use_tc_tiling_on_sc=False),
        out_type=jax.ShapeDtypeStruct((NC, R_PAD, tw), jnp.float32),
        scratch_types=[
            pltpu.VMEM((MAXOPS, SLICE), jnp.int32),
            pltpu.VMEM((MAXOPS, SLICE), jnp.int32),
            pltpu.VMEM((gg + ss, SLICE, tw), jnp.float32),
            pltpu.VMEM_SHARED((R_PAD, tw), jnp.float32),
            pltpu.SemaphoreType.DMA,
            pltpu.SemaphoreType.DMA,
        ],
    )(functools.partial(_sc_body, tw, gg, ss))


def kernel(d_features, g_features, M1_mirna_dis, M2_gene_dis, edge_index,
           W1, W2, W3, W4, Wd, bd, Wg, bg, Ws1, Wn1, b1, Ws2, Wn2, b2):
    e3 = edge_index.reshape(2, NSLICE, SLICE)
    za1 = jnp.zeros((RPT, TW1), jnp.float32)
    za2 = jnp.zeros((RPT, HID), jnp.float32)

    bd2 = bd.reshape(1, EMB)
    bg2 = bg.reshape(1, EMB)
    b12 = b1.reshape(1, HID)
    b22 = b2.reshape(1, HID)

    s1d, t1d = _dis_call(d_features, M1_mirna_dis, Wd, bd2, W1, Ws1, Wn1)
    s1g, t1g = _gen_call(g_features.T, M2_gene_dis.T, Wg, bg2, W4, Ws1, Wn1)
    s1 = jnp.concatenate([s1d, s1g], axis=0)
    t1 = jnp.concatenate([t1d, t1g], axis=0)

    acc1 = _sc_agg_kernel(TW1, 3, 2)(t1, e3, za1)
    t2, s2 = _combine1_call(s1, acc1, b12, Wn2, Ws2)
    acc2 = _sc_agg_kernel(HID, 3, 3)(t2, e3, za2)
    return _combine2_call(s2, acc2, acc1, b22)


# speedup vs baseline: 11.4510x; 1.0008x over previous
import functools

import jax
import jax.numpy as jnp
from jax import lax
from jax.experimental import pallas as pl
from jax.experimental.pallas import tpu as pltpu
from jax.experimental.pallas import tpu_sc as plsc

ND = 4000
NG = 6000
NN = ND + NG
NE = 320000
EMB = 128
HID = 64
TW1 = HID + 16

NC = 2
NS = 16
NW = NC * NS

SLICE = 128

R_PAD = 10240
RPT = R_PAD // NS


def _dis_body(d_ref, m1_ref, wd_ref, bd_ref, w1_ref, ws1_ref, wn1_ref,
              s1_ref, t1_ref):
    rep = 0.9 * (jnp.dot(d_ref[...], wd_ref[...],
                         preferred_element_type=jnp.float32) + bd_ref[...])
    rep = rep + 0.1 * lax.dot_general(
        m1_ref[...], w1_ref[...], (((0,), (0,)), ((), ())),
        preferred_element_type=jnp.float32)
    s1_ref[...] = jnp.dot(rep, ws1_ref[...], preferred_element_type=jnp.float32)
    t1 = jnp.dot(rep, wn1_ref[...], preferred_element_type=jnp.float32)
    t1_ref[...] = jnp.concatenate(
        [t1, jnp.ones((t1.shape[0], 16), jnp.float32)], axis=1)


def _gen_body(gt_ref, m2t_ref, wg_ref, bg_ref, w4_ref, ws1_ref, wn1_ref,
              s1_ref, t1_ref):
    rep = 0.9 * (lax.dot_general(
        gt_ref[...], wg_ref[...], (((0,), (0,)), ((), ())),
        preferred_element_type=jnp.float32) + bg_ref[...])
    rep = rep + 0.1 * lax.dot_general(
        m2t_ref[...], w4_ref[...], (((0,), (0,)), ((), ())),
        preferred_element_type=jnp.float32)
    s1_ref[...] = jnp.dot(rep, ws1_ref[...], preferred_element_type=jnp.float32)
    t1 = jnp.dot(rep, wn1_ref[...], preferred_element_type=jnp.float32)
    t1_ref[...] = jnp.concatenate(
        [t1, jnp.ones((t1.shape[0], 16), jnp.float32)], axis=1)


def _combine1_body(s1_ref, acc_ref, b1_ref, wn2_ref, ws2_ref,
                   t2_ref, s2_ref):
    a = acc_ref[0] + acc_ref[1]
    agg = a[:, :HID]
    deg = a[:, HID:HID + 1]
    hn = agg / jnp.maximum(deg, 1.0)
    h1 = jnp.maximum(s1_ref[...] + hn + b1_ref[...], 0.0)
    t2_ref[...] = jnp.dot(h1, wn2_ref[...], preferred_element_type=jnp.float32)
    s2_ref[...] = jnp.dot(h1, ws2_ref[...], preferred_element_type=jnp.float32)


def _combine2_body(s2_ref, acc_ref, acc1_ref, b2_ref, out_ref):
    agg = acc_ref[0] + acc_ref[1]
    deg = acc1_ref[0, :, HID:HID + 1] + acc1_ref[1, :, HID:HID + 1]
    out_ref[...] = s2_ref[...] + agg / jnp.maximum(deg, 1.0) + b2_ref[...]


def _dis_call(d, m1, wd, bd, w1, ws1, wn1):
    blk = 2048
    grid = (ND + blk - 1) // blk
    return pl.pallas_call(
        _dis_body,
        grid=(grid,),
        in_specs=[
            pl.BlockSpec((blk, 383), lambda i: (i, 0)),
            pl.BlockSpec((2000, blk), lambda i: (0, i)),
            pl.BlockSpec((383, EMB), lambda i: (0, 0)),
            pl.BlockSpec((1, EMB), lambda i: (0, 0)),
            pl.BlockSpec((2000, EMB), lambda i: (0, 0)),
            pl.BlockSpec((EMB, HID), lambda i: (0, 0)),
            pl.BlockSpec((EMB, HID), lambda i: (0, 0)),
        ],
        out_specs=[
            pl.BlockSpec((blk, HID), lambda i: (i, 0)),
            pl.BlockSpec((blk, TW1), lambda i: (i, 0)),
        ],
        out_shape=[
            jax.ShapeDtypeStruct((ND, HID), jnp.float32),
            jax.ShapeDtypeStruct((ND, TW1), jnp.float32),
        ],
    )(d, m1, wd, bd, w1, ws1, wn1)


def _gen_call(gt, m2t, wg, bg, w4, ws1, wn1):
    blk = 512
    grid = (NG + blk - 1) // blk
    return pl.pallas_call(
        _gen_body,
        grid=(grid,),
        in_specs=[
            pl.BlockSpec((4395, blk), lambda i: (0, i)),
            pl.BlockSpec((4000, blk), lambda i: (0, i)),
            pl.BlockSpec((4395, EMB), lambda i: (0, 0)),
            pl.BlockSpec((1, EMB), lambda i: (0, 0)),
            pl.BlockSpec((4000, EMB), lambda i: (0, 0)),
            pl.BlockSpec((EMB, HID), lambda i: (0, 0)),
            pl.BlockSpec((EMB, HID), lambda i: (0, 0)),
        ],
        out_specs=[
            pl.BlockSpec((blk, HID), lambda i: (i, 0)),
            pl.BlockSpec((blk, TW1), lambda i: (i, 0)),
        ],
        out_shape=[
            jax.ShapeDtypeStruct((NG, HID), jnp.float32),
            jax.ShapeDtypeStruct((NG, TW1), jnp.float32),
        ],
    )(gt, m2t, wg, bg, w4, ws1, wn1)


def _combine1_call(s1, acc, b1, wn2, ws2):
    blk = 2560
    grid = (NN + blk - 1) // blk
    return pl.pallas_call(
        _combine1_body,
        grid=(grid,),
        in_specs=[
            pl.BlockSpec((blk, HID), lambda i: (i, 0)),
            pl.BlockSpec((NC, blk, TW1), lambda i: (0, i, 0)),
            pl.BlockSpec((1, HID), lambda i: (0, 0)),
            pl.BlockSpec((HID, HID), lambda i: (0, 0)),
            pl.BlockSpec((HID, HID), lambda i: (0, 0)),
        ],
        out_specs=[
            pl.BlockSpec((blk, HID), lambda i: (i, 0)),
            pl.BlockSpec((blk, HID), lambda i: (i, 0)),
        ],
        out_shape=[
            jax.ShapeDtypeStruct((NN, HID), jnp.float32),
            jax.ShapeDtypeStruct((NN, HID), jnp.float32),
        ],
    )(s1, acc, b1, wn2, ws2)


def _combine2_call(s2, acc2, acc1, b2):
    blk = 2560
    grid = (NN + blk - 1) // blk
    return pl.pallas_call(
        _combine2_body,
        grid=(grid,),
        in_specs=[
            pl.BlockSpec((blk, HID), lambda i: (i, 0)),
            pl.BlockSpec((NC, blk, HID), lambda i: (0, i, 0)),
            pl.BlockSpec((NC, blk, TW1), lambda i: (0, i, 0)),
            pl.BlockSpec((1, HID), lambda i: (0, 0)),
        ],
        out_specs=pl.BlockSpec((blk, HID), lambda i: (i, 0)),
        out_shape=jax.ShapeDtypeStruct((NN, HID), jnp.float32),
    )(s2, acc2, acc1, b2)


NSLICE = NE // SLICE
MAXOPS = 79


def _sc_body(tw, gg, ss, *refs):
    (t_hbm, e_hbm, za_hbm,
     acc_out,
     idx_s, idx_d, rows, acc_sh, sem_g, sem_s) = refs
    depth = gg + ss
    c = lax.axis_index("c")
    s = lax.axis_index("s")
    wid = s * NC + c
    lo = wid * NSLICE // NW
    n = (wid + 1) * NSLICE // NW - lo

    pltpu.sync_copy(za_hbm, acc_sh.at[pl.ds(s * RPT, RPT)])
    pltpu.sync_copy(e_hbm.at[0, pl.ds(lo, MAXOPS)], idx_s)
    pltpu.sync_copy(e_hbm.at[1, pl.ds(lo, MAXOPS)], idx_d)
    plsc.subcore_barrier()

    def gather(u):
        pltpu.async_copy(t_hbm.at[idx_s.at[u]], rows.at[lax.rem(u, depth)],
                         sem_g)

    def drain(sem):
        pltpu.make_async_copy(t_hbm.at[pl.ds(0, SLICE)], rows.at[0],
                              sem).wait()

    for j in range(gg):
        gather(j)

    def step(u, carry):
        @pl.when(u >= ss)
        def _():
            drain(sem_s)

        @pl.when(u + gg <= n - 1)
        def _():
            gather(u + gg)

        drain(sem_g)
        pltpu.async_copy(rows.at[lax.rem(u, depth)],
                         acc_sh.at[idx_d.at[u]], sem_s, add=True)
        return carry

    lax.fori_loop(0, n, step, 0)
    for _ in range(ss):
        drain(sem_s)
    plsc.subcore_barrier()
    pltpu.sync_copy(acc_sh.at[pl.ds(s * RPT, RPT)],
                    acc_out.at[c, pl.ds(s * RPT, RPT)])


@functools.lru_cache(maxsize=None)
def _sc_agg_kernel(tw, gg, ss):
    return functools.partial(
        pl.kernel,
        mesh=plsc.VectorSubcoreMesh(core_axis_name="c", subcore_axis_name="s"),
        compiler_params=pltpu.CompilerParams(use_tc_tiling_on_sc=False),
        out_type=jax.ShapeDtypeStruct((NC, R_PAD, tw), jnp.float32),
        scratch_types=[
            pltpu.VMEM((MAXOPS, SLICE), jnp.int32),
            pltpu.VMEM((MAXOPS, SLICE), jnp.int32),
            pltpu.VMEM((gg + ss, SLICE, tw), jnp.float32),
            pltpu.VMEM_SHARED((R_PAD, tw), jnp.float32),
            pltpu.SemaphoreType.DMA,
            pltpu.SemaphoreType.DMA,
        ],
    )(functools.partial(_sc_body, tw, gg, ss))


def kernel(d_features, g_features, M1_mirna_dis, M2_gene_dis, edge_index,
           W1, W2, W3, W4, Wd, bd, Wg, bg, Ws1, Wn1, b1, Ws2, Wn2, b2):
    e3 = edge_index.reshape(2, NSLICE, SLICE)
    za1 = jnp.zeros((RPT, TW1), jnp.float32)
    za2 = jnp.zeros((RPT, HID), jnp.float32)

    bd2 = bd.reshape(1, EMB)
    bg2 = bg.reshape(1, EMB)
    b12 = b1.reshape(1, HID)
    b22 = b2.reshape(1, HID)

    s1d, t1d = _dis_call(d_features, M1_mirna_dis, Wd, bd2, W1, Ws1, Wn1)
    s1g, t1g = _gen_call(g_features.T, M2_gene_dis.T, Wg, bg2, W4, Ws1, Wn1)
    s1 = jnp.concatenate([s1d, s1g], axis=0)
    t1 = jnp.concatenate([t1d, t1g], axis=0)

    acc1 = _sc_agg_kernel(TW1, 3, 2)(t1, e3, za1)
    t2, s2 = _combine1_call(s1, acc1, b12, Wn2, Ws2)
    acc2 = _sc_agg_kernel(HID, 3, 3)(t2, e3, za2)
    return _combine2_call(s2, acc2, acc1, b22)
